# Initial kernel scaffold; baseline (speedup 1.0000x reference)
import jax
import jax.numpy as jnp
from jax.experimental import pallas as pl


def kernel(x, edge_index, W1, b1, W2, b2, Wc, bc):
    raise NotImplementedError("write your pallas kernel here")


# speedup vs baseline: 13.1468x; 13.1468x over previous
import functools

import jax
import jax.numpy as jnp
from jax import lax
from jax.experimental import pallas as pl
from jax.experimental.pallas import tpu as pltpu
from jax.experimental.pallas import tpu_sc as plsc

N = 10000
E = 320000
F = 128
NC, NS = 2, 16
NW = NC * NS
EPT = E // NW
B = 80
NB = EPT // B
ZR = 624
ZCH = 104
NZ = ZR // ZCH
TAIL = N - NS * ZR

_mesh = plsc.VectorSubcoreMesh(core_axis_name="c", subcore_axis_name="s")


def _deg_body(dst_hbm, out_hbm, zbuf, ones, didx, acc, sem):
    c = lax.axis_index("c")
    s = lax.axis_index("s")
    wid = s * NC + c

    def fill(r, carry):
        zbuf[r, :] = jnp.zeros((16,), jnp.float32)
        ones[r % B, :] = jnp.ones((16,), jnp.float32)
        return carry

    lax.fori_loop(0, ZCH, fill, 0)
    for k in range(NZ):
        pltpu.sync_copy(zbuf, acc.at[pl.ds(s * ZR + k * ZCH, ZCH)])

    @pl.when(s == 0)
    def _():
        pltpu.sync_copy(zbuf.at[pl.ds(0, TAIL)], acc.at[pl.ds(NS * ZR, TAIL)])

    plsc.subcore_barrier()

    base = wid * EPT

    def step(i, carry):
        pltpu.sync_copy(dst_hbm.at[pl.ds(base + i * B, B)], didx)
        pltpu.sync_copy(ones, acc.at[didx], add=True)
        return carry

    lax.fori_loop(0, NB, step, 0)
    plsc.subcore_barrier()
    pltpu.sync_copy(acc.at[pl.ds(s * ZR, ZR)], out_hbm.at[c, pl.ds(s * ZR, ZR)])

    @pl.when(s == 0)
    def _():
        pltpu.sync_copy(acc.at[pl.ds(NS * ZR, TAIL)],
                        out_hbm.at[c, pl.ds(NS * ZR, TAIL)])


_deg = pl.kernel(
    _deg_body,
    out_type=jax.ShapeDtypeStruct((NC, N, 16), jnp.float32),
    mesh=_mesh,
    scratch_types=[
        pltpu.VMEM((ZCH, 16), jnp.float32),
        pltpu.VMEM((B, 16), jnp.float32),
        pltpu.VMEM((B,), jnp.int32),
        pltpu.VMEM_SHARED((N, 16), jnp.float32),
        pltpu.SemaphoreType.DMA,
    ],
)


def _agg_body(y_hbm, src_hbm, dst_hbm, out_hbm, zbuf, sidx, didx, rows, acc, sem):
    c = lax.axis_index("c")
    s = lax.axis_index("s")
    wid = s * NC + c

    def fill(r, carry):
        for j in range(F // 16):
            zbuf[r, pl.ds(j * 16, 16)] = jnp.zeros((16,), jnp.float32)
        return carry

    lax.fori_loop(0, ZCH, fill, 0)
    for k in range(NZ):
        pltpu.sync_copy(zbuf, acc.at[pl.ds(s * ZR + k * ZCH, ZCH)])

    @pl.when(s == 0)
    def _():
        pltpu.sync_copy(zbuf.at[pl.ds(0, TAIL)], acc.at[pl.ds(NS * ZR, TAIL)])

    plsc.subcore_barrier()

    base = wid * EPT

    def step(i, carry):
        off = base + i * B
        pltpu.sync_copy(src_hbm.at[pl.ds(off, B)], sidx)
        pltpu.sync_copy(dst_hbm.at[pl.ds(off, B)], didx)
        pltpu.async_copy(y_hbm.at[sidx], rows, sem).wait()
        pltpu.sync_copy(rows, acc.at[didx], add=True)
        return carry

    lax.fori_loop(0, NB, step, 0)
    plsc.subcore_barrier()
    pltpu.sync_copy(acc.at[pl.ds(s * ZR, ZR)], out_hbm.at[c, pl.ds(s * ZR, ZR)])

    @pl.when(s == 0)
    def _():
        pltpu.sync_copy(acc.at[pl.ds(NS * ZR, TAIL)],
                        out_hbm.at[c, pl.ds(NS * ZR, TAIL)])


_agg = pl.kernel(
    _agg_body,
    out_type=jax.ShapeDtypeStruct((NC, N, F), jnp.float32),
    mesh=_mesh,
    scratch_types=[
        pltpu.VMEM((ZCH, F), jnp.float32),
        pltpu.VMEM((B,), jnp.int32),
        pltpu.VMEM((B,), jnp.int32),
        pltpu.VMEM((B, F), jnp.float32),
        pltpu.VMEM_SHARED((N, F), jnp.float32),
        pltpu.SemaphoreType.DMA,
    ],
)


def _tc1_body(x_ref, w_ref, degp_ref, y_ref, dinv_ref):
    deg = degp_ref[0, :, 0:1] + degp_ref[1, :, 0:1] + 1.0
    dinv = lax.rsqrt(deg)
    xw = jnp.dot(x_ref[...], w_ref[...], preferred_element_type=jnp.float32)
    y_ref[...] = xw * dinv
    dinv_ref[...] = dinv


_tc1 = pl.pallas_call(
    _tc1_body,
    out_shape=[
        jax.ShapeDtypeStruct((N, F), jnp.float32),
        jax.ShapeDtypeStruct((N, 1), jnp.float32),
    ],
)


def _tc2_body(aggp_ref, y_ref, dinv_ref, b_ref, w_ref, y2_ref):
    h = aggp_ref[0] + aggp_ref[1] + y_ref[...]
    h = jnp.maximum(h * dinv_ref[...] + b_ref[...], 0.0)
    y2_ref[...] = jnp.dot(h, w_ref[...], preferred_element_type=jnp.float32) * dinv_ref[...]


_tc2 = pl.pallas_call(
    _tc2_body,
    out_shape=jax.ShapeDtypeStruct((N, F), jnp.float32),
)


def _tc3_body(aggp_ref, y_ref, dinv_ref, b_ref, wc_ref, bc_ref, out_ref):
    h = aggp_ref[0] + aggp_ref[1] + y_ref[...]
    h = jnp.maximum(h * dinv_ref[...] + b_ref[...], 0.0)
    out_ref[...] = jnp.dot(h, wc_ref[...], preferred_element_type=jnp.float32) + bc_ref[...]


_tc3 = pl.pallas_call(
    _tc3_body,
    out_shape=jax.ShapeDtypeStruct((N, 1), jnp.float32),
)


@jax.jit
def kernel(x, edge_index, W1, b1, W2, b2, Wc, bc):
    src = edge_index[0]
    dst = edge_index[1]
    degp = _deg(dst)
    y1, dinv = _tc1(x, W1, degp)
    aggp1 = _agg(y1, src, dst)
    y2 = _tc2(aggp1, y1, dinv, b1.reshape(1, F), W2)
    aggp2 = _agg(y2, src, dst)
    out = _tc3(aggp2, y2, dinv, b2.reshape(1, F), Wc, bc.reshape(1, 1))
    return out[:, 0]
